# baseline (speedup 1.0000x reference)
import jax
import jax.numpy as jnp
from jax.experimental import pallas as pl
from jax.experimental.pallas import tpu as pltpu


def _make_dma_kernel(B, S, D):
    CHUNKS = 8
    CS = S // CHUNKS

    def _kern(tid_ref, x_hbm, p_hbm, o_hbm, sem_big, sem_rows):
        copies = []
        for b in range(B):
            for ci in range(CHUNKS):
                c = pltpu.make_async_copy(
                    x_hbm.at[b, pl.ds(ci * CS, CS), :],
                    o_hbm.at[b, pl.ds(ci * CS, CS), :],
                    sem_big,
                )
                c.start()
                copies.append(c)
        for b in range(B):
            c = pltpu.make_async_copy(
                p_hbm.at[pl.ds(tid_ref[b], 1), :],
                o_hbm.at[b, pl.ds(S, 1), :],
                sem_rows,
            )
            c.start()
            copies.append(c)
        for c in copies:
            c.wait()

    return _kern


def kernel(x, task_id, prompt):
    B, S, D = x.shape
    task_id32 = task_id.astype(jnp.int32)

    out = pl.pallas_call(
        _make_dma_kernel(B, S, D),
        in_specs=[
            pl.BlockSpec(memory_space=pltpu.MemorySpace.SMEM),
            pl.BlockSpec(memory_space=pltpu.MemorySpace.HBM),
            pl.BlockSpec(memory_space=pltpu.MemorySpace.HBM),
        ],
        out_specs=pl.BlockSpec(memory_space=pltpu.MemorySpace.HBM),
        out_shape=jax.ShapeDtypeStruct((B, S + 1, D), x.dtype),
        scratch_shapes=[pltpu.SemaphoreType.DMA, pltpu.SemaphoreType.DMA],
    )(task_id32, x, prompt)
    return (out, task_id)


# speedup vs baseline: 11.4872x; 11.4872x over previous
import functools

import jax
import jax.numpy as jnp
from jax.experimental import pallas as pl
from jax.experimental.pallas import tpu as pltpu

_BS = 256


def _concat_kernel(tid_ref, x_ref, p_ref, o_ref, *, ns):
    s = pl.program_id(1)

    @pl.when(s < ns)
    def _copy():
        o_ref[...] = x_ref[...]

    @pl.when(s == ns)
    def _row():
        o_ref[0, 0, :] = p_ref[0, 0]


def kernel(x, task_id, prompt):
    B, S, D = x.shape
    ns = S // _BS
    task_id32 = task_id.astype(jnp.int32)
    prompt3 = prompt.reshape(prompt.shape[0], 1, prompt.shape[1])

    grid_spec = pltpu.PrefetchScalarGridSpec(
        num_scalar_prefetch=1,
        grid=(B, ns + 1),
        in_specs=[
            pl.BlockSpec((1, _BS, D), lambda b, s, tid: (b, jnp.minimum(s, ns - 1), 0)),
            pl.BlockSpec((1, 1, D), lambda b, s, tid: (tid[b], 0, 0)),
        ],
        out_specs=pl.BlockSpec((1, _BS, D), lambda b, s, tid: (b, s, 0)),
    )

    out = pl.pallas_call(
        functools.partial(_concat_kernel, ns=ns),
        grid_spec=grid_spec,
        out_shape=jax.ShapeDtypeStruct((B, S + 1, D), x.dtype),
    )(task_id32, x, prompt3)
    return (out, task_id)


# speedup vs baseline: 12.4618x; 1.0848x over previous
import functools

import jax
import jax.numpy as jnp
from jax.experimental import pallas as pl
from jax.experimental.pallas import tpu as pltpu

_BS = 512


def _copy_kernel(x_ref, o_ref):
    o_ref[...] = x_ref[...]


def _row_kernel(tid_ref, buf_ref, p_ref, o_ref):
    o_ref[...] = p_ref[...]


def kernel(x, task_id, prompt):
    B, S, D = x.shape
    ns = S // _BS
    task_id32 = task_id.astype(jnp.int32)
    x4 = x.reshape(B, S, 1, D)
    prompt4 = prompt.reshape(prompt.shape[0], 1, 1, prompt.shape[1])

    buf = pl.pallas_call(
        _copy_kernel,
        grid=(B, ns),
        in_specs=[pl.BlockSpec((1, _BS, 1, D), lambda b, s: (b, s, 0, 0))],
        out_specs=pl.BlockSpec((1, _BS, 1, D), lambda b, s: (b, s, 0, 0)),
        out_shape=jax.ShapeDtypeStruct((B, S + 1, 1, D), x.dtype),
    )(x4)

    grid_spec = pltpu.PrefetchScalarGridSpec(
        num_scalar_prefetch=1,
        grid=(B,),
        in_specs=[
            pl.BlockSpec(memory_space=pltpu.MemorySpace.HBM),
            pl.BlockSpec((1, 1, 1, D), lambda b, tid: (tid[b], 0, 0, 0)),
        ],
        out_specs=pl.BlockSpec((1, 1, 1, D), lambda b, tid: (b, S, 0, 0)),
    )

    out4 = pl.pallas_call(
        _row_kernel,
        grid_spec=grid_spec,
        out_shape=jax.ShapeDtypeStruct((B, S + 1, 1, D), x.dtype),
        input_output_aliases={1: 0},
    )(task_id32, buf, prompt4)

    return (out4.reshape(B, S + 1, D), task_id)


# speedup vs baseline: 12.4855x; 1.0019x over previous
import functools

import jax
import jax.numpy as jnp
from jax.experimental import pallas as pl
from jax.experimental.pallas import tpu as pltpu

_BS = 512


def _copy_kernel(x_ref, o_ref):
    o_ref[...] = x_ref[...]


def _row_kernel(tid_ref, buf_ref, p_ref, o_ref):
    o_ref[...] = p_ref[...]


def kernel(x, task_id, prompt):
    B, S, D = x.shape
    ns = S // _BS
    task_id32 = task_id.astype(jnp.int32)
    x4 = x.reshape(B, S, 1, D)
    prompt4 = prompt.reshape(prompt.shape[0], 1, 1, prompt.shape[1])

    buf = pl.pallas_call(
        _copy_kernel,
        grid=(B, ns),
        in_specs=[pl.BlockSpec((1, _BS, 1, D), lambda b, s: (b, s, 0, 0))],
        out_specs=pl.BlockSpec((1, _BS, 1, D), lambda b, s: (b, s, 0, 0)),
        out_shape=jax.ShapeDtypeStruct((B, S + 1, 1, D), x.dtype),
        compiler_params=pltpu.CompilerParams(
            dimension_semantics=("parallel", "parallel")
        ),
    )(x4)

    grid_spec = pltpu.PrefetchScalarGridSpec(
        num_scalar_prefetch=1,
        grid=(B,),
        in_specs=[
            pl.BlockSpec(memory_space=pltpu.MemorySpace.HBM),
            pl.BlockSpec((1, 1, 1, D), lambda b, tid: (tid[b], 0, 0, 0)),
        ],
        out_specs=pl.BlockSpec((1, 1, 1, D), lambda b, tid: (b, S, 0, 0)),
    )

    out4 = pl.pallas_call(
        _row_kernel,
        grid_spec=grid_spec,
        out_shape=jax.ShapeDtypeStruct((B, S + 1, 1, D), x.dtype),
        input_output_aliases={1: 0},
    )(task_id32, buf, prompt4)

    return (out4.reshape(B, S + 1, D), task_id)


# speedup vs baseline: 12.9077x; 1.0338x over previous
import functools

import jax
import jax.numpy as jnp
from jax.experimental import pallas as pl
from jax.experimental.pallas import tpu as pltpu

_BS = 2048


def _copy_kernel(x_ref, o_ref):
    o_ref[...] = x_ref[...]


def _row_kernel(tid_ref, buf_ref, p_ref, o_ref):
    o_ref[...] = p_ref[...]


def kernel(x, task_id, prompt):
    B, S, D = x.shape
    ns = S // _BS
    task_id32 = task_id.astype(jnp.int32)
    x4 = x.reshape(B, S, 1, D)
    prompt4 = prompt.reshape(prompt.shape[0], 1, 1, prompt.shape[1])

    buf = pl.pallas_call(
        _copy_kernel,
        grid=(B, ns),
        in_specs=[pl.BlockSpec((1, _BS, 1, D), lambda b, s: (b, s, 0, 0))],
        out_specs=pl.BlockSpec((1, _BS, 1, D), lambda b, s: (b, s, 0, 0)),
        out_shape=jax.ShapeDtypeStruct((B, S + 1, 1, D), x.dtype),
        compiler_params=pltpu.CompilerParams(
            dimension_semantics=("parallel", "parallel")
        ),
    )(x4)

    grid_spec = pltpu.PrefetchScalarGridSpec(
        num_scalar_prefetch=1,
        grid=(B,),
        in_specs=[
            pl.BlockSpec(memory_space=pltpu.MemorySpace.HBM),
            pl.BlockSpec((1, 1, 1, D), lambda b, tid: (tid[b], 0, 0, 0)),
        ],
        out_specs=pl.BlockSpec((1, 1, 1, D), lambda b, tid: (b, S, 0, 0)),
    )

    out4 = pl.pallas_call(
        _row_kernel,
        grid_spec=grid_spec,
        out_shape=jax.ShapeDtypeStruct((B, S + 1, 1, D), x.dtype),
        input_output_aliases={1: 0},
    )(task_id32, buf, prompt4)

    return (out4.reshape(B, S + 1, D), task_id)
